# Initial kernel scaffold; baseline (speedup 1.0000x reference)
import jax
import jax.numpy as jnp
from jax.experimental import pallas as pl


def kernel(x, dw1, pw1, dw2, pw2, dw3, pw3, dw4, pw4):
    raise NotImplementedError("write your pallas kernel here")


# speedup vs baseline: 1.5665x; 1.5665x over previous
import jax
import jax.numpy as jnp
from jax.experimental import pallas as pl
from jax.experimental.pallas import tpu as pltpu

H = W = 28
HW = H * W
C = 512
PATCH = 7
DISP = 3
PAD = 128


def _silu(v):
    return v * jax.nn.sigmoid(v)


def _xcoord():
    return jax.lax.broadcasted_iota(jnp.int32, (1, HW), 1) % W


def _shift_mask(dx, xc):
    valid = jnp.logical_and(xc + dx >= 0, xc + dx < W)
    return valid.astype(jnp.float32)


def _flow_kernel(xa_ref, xb_ref, out_ref):
    f1 = xa_ref[0, :, 0, 0, :]
    f2 = xb_ref[0, :, 0, 0, :]
    n1 = jnp.sum(f1 * f1, axis=0, keepdims=True) + 1e-6
    f1 = f1 * jax.lax.rsqrt(n1)
    n2 = jnp.sum(f2 * f2, axis=0, keepdims=True) + 1e-6
    f2 = f2 * jax.lax.rsqrt(n2)

    zpad = jnp.zeros((C, PAD), jnp.float32)
    f2p = jnp.concatenate([zpad, f2, zpad], axis=1)

    xc = _xcoord()
    rows = []
    for d in range(PATCH * PATCH):
        dy = d // PATCH - DISP
        dx = d % PATCH - DISP
        off = PAD + dy * W + dx
        s = f2p[:, off:off + HW]
        cd = jnp.sum(f1 * s, axis=0, keepdims=True)
        rows.append(cd * _shift_mask(dx, xc))
    corr = jnp.concatenate(rows, axis=0)

    m = _silu(corr)
    topv = jnp.max(m, axis=0, keepdims=True)
    di = jax.lax.broadcasted_iota(jnp.int32, (PATCH * PATCH, HW), 0)
    idx = jnp.min(jnp.where(m == topv, di, PATCH * PATCH), axis=0,
                  keepdims=True)
    idx_y = (idx // PATCH).astype(jnp.float32)
    idx_x = (idx % PATCH).astype(jnp.float32)
    gy = (di // PATCH).astype(jnp.float32)
    gx = (di % PATCH).astype(jnp.float32)
    gauss = jnp.exp(-((gx - idx_x) ** 2 + (gy - idx_y) ** 2) / 50.0)
    m = gauss * m * 100.0
    mmax = jnp.max(m, axis=0, keepdims=True)
    e = jnp.exp(m - mmax)
    s = e / jnp.sum(e, axis=0, keepdims=True)
    flow_x = jnp.sum(s * (gx - DISP), axis=0, keepdims=True) / float(DISP)
    flow_y = jnp.sum(s * (gy - DISP), axis=0, keepdims=True) / float(DISP)
    out_ref[0] = jnp.concatenate([flow_x, flow_y, topv], axis=0)


def _dwconv(xin, wk, k, xc):
    c = xin.shape[0]
    p = (k - 1) // 2
    zpad = jnp.zeros((c, PAD), jnp.float32)
    xp = jnp.concatenate([zpad, xin, zpad], axis=1)
    acc = jnp.zeros((c, HW), jnp.float32)
    for t in range(k * k):
        ky = t // k - p
        kx = t % k - p
        s = xp[:, PAD + ky * W + kx:PAD + ky * W + kx + HW]
        acc = acc + s * wk[:, t:t + 1] * _shift_mask(kx, xc)
    return acc


def _refine_kernel(x1_ref, x_ref, dw1r, pw1r, dw2r, pw2r, dw3r, pw3r,
                   dw4r, pw4r, out_ref):
    xc = _xcoord()
    a = x1_ref[0]
    a = _silu(_dwconv(a, dw1r[...], 5, xc))
    a = _silu(jnp.dot(pw1r[...], a, preferred_element_type=jnp.float32))
    a = _silu(_dwconv(a, dw2r[...], 3, xc))
    a = _silu(jnp.dot(pw2r[...], a, preferred_element_type=jnp.float32))
    a = _silu(_dwconv(a, dw3r[...], 3, xc))
    a = _silu(jnp.dot(pw3r[...], a, preferred_element_type=jnp.float32))
    a = _silu(_dwconv(a, dw4r[...], 3, xc))
    a = _silu(jnp.dot(pw4r[...], a, preferred_element_type=jnp.float32))
    out_ref[0, :, 0, 0, :] = a + x_ref[0, :, 0, 0, :]


@jax.jit
def kernel(x, dw1, pw1, dw2, pw2, dw3, pw3, dw4, pw4):
    b, c, t, h, w = x.shape
    nt = b * (t - 1)
    nf = b * t
    xr = x.reshape(b, c, t, 1, h * w)

    frame_spec = lambda imap: pl.BlockSpec((1, c, 1, 1, h * w), imap)
    flow = pl.pallas_call(
        _flow_kernel,
        grid=(nt,),
        in_specs=[
            frame_spec(lambda i: (i // (t - 1), 0, i % (t - 1), 0, 0)),
            frame_spec(lambda i: (i // (t - 1), 0, i % (t - 1) + 1, 0, 0)),
        ],
        out_specs=pl.BlockSpec((1, 3, h * w), lambda i: (i, 0, 0)),
        out_shape=jax.ShapeDtypeStruct((nt, 3, h * w), jnp.float32),
        compiler_params=pltpu.CompilerParams(
            dimension_semantics=("parallel",)),
    )(xr, xr)

    fr = flow.reshape(b, t - 1, 3, h * w)
    x1 = jnp.concatenate([fr, fr[:, -1:]], axis=1).reshape(nf, 3, h * w)

    wfull = lambda a: pl.BlockSpec(a.shape, lambda i: (0,) * a.ndim)
    dw1r = dw1.reshape(3, 25)
    dw2r = dw2.reshape(16, 9)
    dw3r = dw3.reshape(32, 9)
    dw4r = dw4.reshape(64, 9)
    pw1r = pw1.reshape(16, 3)
    pw2r = pw2.reshape(32, 16)
    pw3r = pw3.reshape(64, 32)
    pw4r = pw4.reshape(512, 64)

    out = pl.pallas_call(
        _refine_kernel,
        grid=(nf,),
        in_specs=[
            pl.BlockSpec((1, 3, h * w), lambda i: (i, 0, 0)),
            frame_spec(lambda i: (i // t, 0, i % t, 0, 0)),
            wfull(dw1r), wfull(pw1r), wfull(dw2r), wfull(pw2r),
            wfull(dw3r), wfull(pw3r), wfull(dw4r), wfull(pw4r),
        ],
        out_specs=frame_spec(lambda i: (i // t, 0, i % t, 0, 0)),
        out_shape=jax.ShapeDtypeStruct((b, c, t, 1, h * w), jnp.float32),
        compiler_params=pltpu.CompilerParams(
            dimension_semantics=("parallel",)),
    )(x1, xr, dw1r, pw1r, dw2r, pw2r, dw3r, pw3r, dw4r, pw4r)

    return out.reshape(b, c, t, h, w)


# speedup vs baseline: 1.7641x; 1.1262x over previous
import jax
import jax.numpy as jnp
from jax.experimental import pallas as pl
from jax.experimental.pallas import tpu as pltpu

H = W = 28
HW = H * W
C = 512
PATCH = 7
DISP = 3
PAD = 128


def _silu(v):
    return v * jax.nn.sigmoid(v)


def _xcoord():
    return jax.lax.broadcasted_iota(jnp.int32, (1, HW), 1) % W


def _shift_mask(dx, xc):
    valid = jnp.logical_and(xc + dx >= 0, xc + dx < W)
    return valid.astype(jnp.float32)


def _flow_kernel(xa_ref, xb_ref, out_ref):
    f1 = xa_ref[0, :, 0, 0, :]
    f2 = xb_ref[0, :, 0, 0, :]
    n1 = jnp.sum(f1 * f1, axis=0, keepdims=True) + 1e-6
    f1 = f1 * jax.lax.rsqrt(n1)
    n2 = jnp.sum(f2 * f2, axis=0, keepdims=True) + 1e-6
    f2 = f2 * jax.lax.rsqrt(n2)

    zpad = jnp.zeros((C, PAD), jnp.float32)
    f1p = jnp.concatenate([zpad, f1, zpad], axis=1)
    f2p = jnp.concatenate([zpad, f2, zpad], axis=1)

    a_dx = [f1p[:, PAD - dx:PAD - dx + HW] for dx in range(-DISP, DISP + 1)]
    b_dy = [f2p[:, PAD + W * dy:PAD + W * dy + HW]
            for dy in range(-DISP, DISP + 1)]

    xc = _xcoord()
    zrow = jnp.zeros((1, PAD), jnp.float32)
    rows = []
    for d in range(PATCH * PATCH):
        dy = d // PATCH - DISP
        dx = d % PATCH - DISP
        r = jnp.sum(a_dx[dx + DISP] * b_dy[dy + DISP], axis=0, keepdims=True)
        rp = jnp.concatenate([zrow, r, zrow], axis=1)
        rows.append(rp[:, PAD + dx:PAD + dx + HW] * _shift_mask(dx, xc))
    corr = jnp.concatenate(rows, axis=0)

    m = _silu(corr)
    topv = jnp.max(m, axis=0, keepdims=True)
    di = jax.lax.broadcasted_iota(jnp.int32, (PATCH * PATCH, HW), 0)
    idx = jnp.min(jnp.where(m == topv, di, PATCH * PATCH), axis=0,
                  keepdims=True)
    idx_y = (idx // PATCH).astype(jnp.float32)
    idx_x = (idx % PATCH).astype(jnp.float32)
    gy = (di // PATCH).astype(jnp.float32)
    gx = (di % PATCH).astype(jnp.float32)
    gauss = jnp.exp(-((gx - idx_x) ** 2 + (gy - idx_y) ** 2) / 50.0)
    m = gauss * m * 100.0
    mmax = jnp.max(m, axis=0, keepdims=True)
    e = jnp.exp(m - mmax)
    s = e / jnp.sum(e, axis=0, keepdims=True)
    flow_x = jnp.sum(s * (gx - DISP), axis=0, keepdims=True) / float(DISP)
    flow_y = jnp.sum(s * (gy - DISP), axis=0, keepdims=True) / float(DISP)
    out_ref[0] = jnp.concatenate([flow_x, flow_y, topv], axis=0)


def _dwconv(xin, wk, k, xc):
    c = xin.shape[0]
    p = (k - 1) // 2
    zpad = jnp.zeros((c, PAD), jnp.float32)
    xp = jnp.concatenate([zpad, xin, zpad], axis=1)
    acc = jnp.zeros((c, HW), jnp.float32)
    for t in range(k * k):
        ky = t // k - p
        kx = t % k - p
        s = xp[:, PAD + ky * W + kx:PAD + ky * W + kx + HW]
        acc = acc + s * wk[:, t:t + 1] * _shift_mask(kx, xc)
    return acc


def _refine_kernel(x1_ref, x_ref, dw1r, pw1r, dw2r, pw2r, dw3r, pw3r,
                   dw4r, pw4r, out_ref):
    xc = _xcoord()
    a = x1_ref[0]
    a = _silu(_dwconv(a, dw1r[...], 5, xc))
    a = _silu(jnp.dot(pw1r[...], a, preferred_element_type=jnp.float32))
    a = _silu(_dwconv(a, dw2r[...], 3, xc))
    a = _silu(jnp.dot(pw2r[...], a, preferred_element_type=jnp.float32))
    a = _silu(_dwconv(a, dw3r[...], 3, xc))
    a = _silu(jnp.dot(pw3r[...], a, preferred_element_type=jnp.float32))
    a = _silu(_dwconv(a, dw4r[...], 3, xc))
    a = _silu(jnp.dot(pw4r[...], a, preferred_element_type=jnp.float32))
    out_ref[0, :, 0, 0, :] = a + x_ref[0, :, 0, 0, :]


@jax.jit
def kernel(x, dw1, pw1, dw2, pw2, dw3, pw3, dw4, pw4):
    b, c, t, h, w = x.shape
    nt = b * (t - 1)
    nf = b * t
    xr = x.reshape(b, c, t, 1, h * w)

    frame_spec = lambda imap: pl.BlockSpec((1, c, 1, 1, h * w), imap)
    flow = pl.pallas_call(
        _flow_kernel,
        grid=(nt,),
        in_specs=[
            frame_spec(lambda i: (i // (t - 1), 0, i % (t - 1), 0, 0)),
            frame_spec(lambda i: (i // (t - 1), 0, i % (t - 1) + 1, 0, 0)),
        ],
        out_specs=pl.BlockSpec((1, 3, h * w), lambda i: (i, 0, 0)),
        out_shape=jax.ShapeDtypeStruct((nt, 3, h * w), jnp.float32),
        compiler_params=pltpu.CompilerParams(
            dimension_semantics=("parallel",)),
    )(xr, xr)

    fr = flow.reshape(b, t - 1, 3, h * w)
    x1 = jnp.concatenate([fr, fr[:, -1:]], axis=1).reshape(nf, 3, h * w)

    wfull = lambda a: pl.BlockSpec(a.shape, lambda i: (0,) * a.ndim)
    dw1r = dw1.reshape(3, 25)
    dw2r = dw2.reshape(16, 9)
    dw3r = dw3.reshape(32, 9)
    dw4r = dw4.reshape(64, 9)
    pw1r = pw1.reshape(16, 3)
    pw2r = pw2.reshape(32, 16)
    pw3r = pw3.reshape(64, 32)
    pw4r = pw4.reshape(512, 64)

    out = pl.pallas_call(
        _refine_kernel,
        grid=(nf,),
        in_specs=[
            pl.BlockSpec((1, 3, h * w), lambda i: (i, 0, 0)),
            frame_spec(lambda i: (i // t, 0, i % t, 0, 0)),
            wfull(dw1r), wfull(pw1r), wfull(dw2r), wfull(pw2r),
            wfull(dw3r), wfull(pw3r), wfull(dw4r), wfull(pw4r),
        ],
        out_specs=frame_spec(lambda i: (i // t, 0, i % t, 0, 0)),
        out_shape=jax.ShapeDtypeStruct((b, c, t, 1, h * w), jnp.float32),
        compiler_params=pltpu.CompilerParams(
            dimension_semantics=("parallel",)),
    )(x1, xr, dw1r, pw1r, dw2r, pw2r, dw3r, pw3r, dw4r, pw4r)

    return out.reshape(b, c, t, h, w)


# speedup vs baseline: 2.7311x; 1.5482x over previous
import jax
import jax.numpy as jnp
from jax.experimental import pallas as pl
from jax.experimental.pallas import tpu as pltpu

H = W = 28
HW = H * W
C = 512
PATCH = 7
DISP = 3
PAD = 128


def _silu(v):
    return v * jax.nn.sigmoid(v)


def _xcoord():
    return jax.lax.broadcasted_iota(jnp.int32, (1, HW), 1) % W


def _shift_mask(dx, xc):
    valid = jnp.logical_and(xc + dx >= 0, xc + dx < W)
    return valid.astype(jnp.float32)


def _flow_kernel(xa_ref, xb_ref, out_ref, f1p_ref, f2p_ref, a_ref, b_ref):
    f1 = xa_ref[0, :, 0, 0, :]
    f2 = xb_ref[0, :, 0, 0, :]
    zpad = jnp.zeros((C, PAD), jnp.float32)
    n1 = jnp.sum(f1 * f1, axis=0, keepdims=True) + 1e-6
    f1p_ref[:, :PAD] = zpad
    f1p_ref[:, PAD:PAD + HW] = f1 * jax.lax.rsqrt(n1)
    f1p_ref[:, PAD + HW:] = zpad
    n2 = jnp.sum(f2 * f2, axis=0, keepdims=True) + 1e-6
    f2p_ref[:, :PAD] = zpad
    f2p_ref[:, PAD:PAD + HW] = f2 * jax.lax.rsqrt(n2)
    f2p_ref[:, PAD + HW:] = zpad

    for dx in range(-DISP, DISP + 1):
        a_ref[dx + DISP] = f1p_ref[:, PAD - dx:PAD - dx + HW]
    for dy in range(-DISP, DISP + 1):
        b_ref[dy + DISP] = f2p_ref[:, PAD + W * dy:PAD + W * dy + HW]

    xc = _xcoord()
    zrow = jnp.zeros((1, PAD), jnp.float32)
    rows = []
    for d in range(PATCH * PATCH):
        dy = d // PATCH - DISP
        dx = d % PATCH - DISP
        r = jnp.sum(a_ref[dx + DISP] * b_ref[dy + DISP], axis=0,
                    keepdims=True)
        rp = jnp.concatenate([zrow, r, zrow], axis=1)
        rows.append(rp[:, PAD + dx:PAD + dx + HW] * _shift_mask(dx, xc))
    corr = jnp.concatenate(rows, axis=0)

    m = _silu(corr)
    topv = jnp.max(m, axis=0, keepdims=True)
    di = jax.lax.broadcasted_iota(jnp.int32, (PATCH * PATCH, HW), 0)
    idx = jnp.min(jnp.where(m == topv, di, PATCH * PATCH), axis=0,
                  keepdims=True)
    idx_y = (idx // PATCH).astype(jnp.float32)
    idx_x = (idx % PATCH).astype(jnp.float32)
    gy = (di // PATCH).astype(jnp.float32)
    gx = (di % PATCH).astype(jnp.float32)
    gauss = jnp.exp(-((gx - idx_x) ** 2 + (gy - idx_y) ** 2) / 50.0)
    m = gauss * m * 100.0
    mmax = jnp.max(m, axis=0, keepdims=True)
    e = jnp.exp(m - mmax)
    s = e / jnp.sum(e, axis=0, keepdims=True)
    flow_x = jnp.sum(s * (gx - DISP), axis=0, keepdims=True) / float(DISP)
    flow_y = jnp.sum(s * (gy - DISP), axis=0, keepdims=True) / float(DISP)
    out_ref[0] = jnp.concatenate([flow_x, flow_y, topv], axis=0)


def _dwconv(xin, wk, k, xc):
    c = xin.shape[0]
    p = (k - 1) // 2
    zpad = jnp.zeros((c, PAD), jnp.float32)
    xp = jnp.concatenate([zpad, xin, zpad], axis=1)
    acc = jnp.zeros((c, HW), jnp.float32)
    for t in range(k * k):
        ky = t // k - p
        kx = t % k - p
        s = xp[:, PAD + ky * W + kx:PAD + ky * W + kx + HW]
        acc = acc + s * wk[:, t:t + 1] * _shift_mask(kx, xc)
    return acc


def _refine_kernel(x1_ref, x_ref, dw1r, pw1r, dw2r, pw2r, dw3r, pw3r,
                   dw4r, pw4r, out_ref):
    xc = _xcoord()
    a = x1_ref[0]
    a = _silu(_dwconv(a, dw1r[...], 5, xc))
    a = _silu(jnp.dot(pw1r[...], a, preferred_element_type=jnp.float32))
    a = _silu(_dwconv(a, dw2r[...], 3, xc))
    a = _silu(jnp.dot(pw2r[...], a, preferred_element_type=jnp.float32))
    a = _silu(_dwconv(a, dw3r[...], 3, xc))
    a = _silu(jnp.dot(pw3r[...], a, preferred_element_type=jnp.float32))
    a = _silu(_dwconv(a, dw4r[...], 3, xc))
    a = _silu(jnp.dot(pw4r[...], a, preferred_element_type=jnp.float32))
    out_ref[0, :, 0, 0, :] = a + x_ref[0, :, 0, 0, :]


@jax.jit
def kernel(x, dw1, pw1, dw2, pw2, dw3, pw3, dw4, pw4):
    b, c, t, h, w = x.shape
    nt = b * (t - 1)
    nf = b * t
    xr = x.reshape(b, c, t, 1, h * w)

    frame_spec = lambda imap: pl.BlockSpec((1, c, 1, 1, h * w), imap)
    flow = pl.pallas_call(
        _flow_kernel,
        grid=(nt,),
        in_specs=[
            frame_spec(lambda i: (i // (t - 1), 0, i % (t - 1), 0, 0)),
            frame_spec(lambda i: (i // (t - 1), 0, i % (t - 1) + 1, 0, 0)),
        ],
        out_specs=pl.BlockSpec((1, 3, h * w), lambda i: (i, 0, 0)),
        out_shape=jax.ShapeDtypeStruct((nt, 3, h * w), jnp.float32),
        scratch_shapes=[
            pltpu.VMEM((C, HW + 2 * PAD), jnp.float32),
            pltpu.VMEM((C, HW + 2 * PAD), jnp.float32),
            pltpu.VMEM((PATCH, C, HW), jnp.float32),
            pltpu.VMEM((PATCH, C, HW), jnp.float32),
        ],
        compiler_params=pltpu.CompilerParams(
            dimension_semantics=("parallel",)),
    )(xr, xr)

    fr = flow.reshape(b, t - 1, 3, h * w)
    x1 = jnp.concatenate([fr, fr[:, -1:]], axis=1).reshape(nf, 3, h * w)

    wfull = lambda a: pl.BlockSpec(a.shape, lambda i: (0,) * a.ndim)
    dw1r = dw1.reshape(3, 25)
    dw2r = dw2.reshape(16, 9)
    dw3r = dw3.reshape(32, 9)
    dw4r = dw4.reshape(64, 9)
    pw1r = pw1.reshape(16, 3)
    pw2r = pw2.reshape(32, 16)
    pw3r = pw3.reshape(64, 32)
    pw4r = pw4.reshape(512, 64)

    out = pl.pallas_call(
        _refine_kernel,
        grid=(nf,),
        in_specs=[
            pl.BlockSpec((1, 3, h * w), lambda i: (i, 0, 0)),
            frame_spec(lambda i: (i // t, 0, i % t, 0, 0)),
            wfull(dw1r), wfull(pw1r), wfull(dw2r), wfull(pw2r),
            wfull(dw3r), wfull(pw3r), wfull(dw4r), wfull(pw4r),
        ],
        out_specs=frame_spec(lambda i: (i // t, 0, i % t, 0, 0)),
        out_shape=jax.ShapeDtypeStruct((b, c, t, 1, h * w), jnp.float32),
        compiler_params=pltpu.CompilerParams(
            dimension_semantics=("parallel",)),
    )(x1, xr, dw1r, pw1r, dw2r, pw2r, dw3r, pw3r, dw4r, pw4r)

    return out.reshape(b, c, t, h, w)
